# baseline (speedup 1.0000x reference)
import functools

import jax
import jax.numpy as jnp
from jax import lax
from jax.experimental import pallas as pl
from jax.experimental.pallas import tpu as pltpu
from jax.experimental.pallas import tpu_sc as plsc

N_USER = 25000
N_OVERLAP = 10000
EMB = 64
LAYERS = 3
N = 50000
BATCH = 4096

NNZ = 800000
HALF = 32
CHUNK = 128
NS = 16
NC = 2
SUP = 2
NSTEP = 200
NCT = SUP * NSTEP
NNZ_PAD = NS * NCT * CHUNK
NROWS2D = NNZ_PAD // CHUNK
RPT = 3000
CCH = 200
SPT = BATCH // NS

_MESH = plsc.VectorSubcoreMesh(core_axis_name="c", subcore_axis_name="s",
                               num_cores=NC, num_subcores=NS)
_SC_PARAMS = pltpu.CompilerParams(use_tc_tiling_on_sc=False)


def _zero_acc(zeros_hbm, acc, t):
    pltpu.sync_copy(zeros_hbm, acc.at[pl.ds(t * RPT, RPT)])

    @pl.when(t < 2)
    def _():
        rx = NS * RPT + t * 1000
        pltpu.sync_copy(zeros_hbm.at[pl.ds(0, 1000)], acc.at[pl.ds(rx, 1000)])


def _spmm_phase(kc, t, ego_hbm, pk2, vals2, zeros_hbm,
                acc, pb, vb, db, isems, gsems, ssems):
    def idx_load(p, s):
        row = t * NCT + s * SUP
        return [
            pltpu.async_copy(pk2.at[kc, pl.ds(row, SUP)], pb.at[p], isems[p]),
            pltpu.async_copy(vals2.at[pl.ds(row, SUP)], vb.at[p], isems[p]),
        ]

    def gathers(p):
        for b in range(SUP):
            pltpu.async_copy(ego_hbm.at[pb.at[p, b, 0]],
                             db.at[p, pl.ds(b * CHUNK, CHUNK)], gsems[p])

    def drain(sem):
        pltpu.make_async_copy(zeros_hbm.at[pl.ds(0, SUP * CHUNK)],
                              db.at[0], sem).wait()

    def scale(p):
        def grp(g, _):
            b = g // (CHUNK // 32)
            o = (g % (CHUNK // 32)) * 32
            for half in range(2):
                v = vb[p, b, pl.ds(o + half * 16, 16)]
                for j in range(16):
                    e = b * CHUNK + o + half * 16 + j
                    s = v[j]
                    db[p, e, pl.ds(0, 16)] = db[p, e, pl.ds(0, 16)] * s
                    db[p, e, pl.ds(16, 16)] = db[p, e, pl.ds(16, 16)] * s
            return 0
        lax.fori_loop(0, SUP * CHUNK // 32, grp, 0)

    def scatters(p):
        for b in range(SUP):
            pltpu.async_copy(db.at[p, pl.ds(b * CHUNK, CHUNK)],
                             acc.at[pb.at[p, b, 1]], ssems[p], add=True)

    for d in idx_load(0, 0):
        d.wait()
    gathers(0)
    for d in idx_load(1, 1):
        d.wait()

    def body2(s, _):
        @pl.when(s > 0)
        def _():
            drain(ssems[1])
        gathers(1)
        drain(gsems[0])
        di = idx_load(0, s + 2)
        scale(0)
        scatters(0)
        for d in di:
            d.wait()
        drain(ssems[0])
        gathers(0)
        drain(gsems[1])
        di = idx_load(1, s + 3)
        scale(1)
        scatters(1)
        for d in di:
            d.wait()
        return 0
    lax.fori_loop(0, (NSTEP - 2) // 2, lambda i, c: body2(i * 2, c), 0)

    drain(ssems[1])
    gathers(1)
    drain(gsems[0])
    scale(0)
    scatters(0)
    drain(gsems[1])
    scale(1)
    scatters(1)
    drain(ssems[0])
    drain(ssems[1])


def _ew_loop(dst, a, b, nrows, f):
    def body(i, _):
        r = i // (HALF // 16)
        h = (i % (HALF // 16)) * 16
        dst[r, pl.ds(h, 16)] = f(a[r, pl.ds(h, 16)], b[r, pl.ds(h, 16)])
        return 0
    lax.fori_loop(0, nrows * HALF // 16, body, 0)


def _combine_chunk(r, kc, layer, acc, src, sum_in, gcf_out, sum_out,
                   db0, db1, db2):
    ar = kc * N + r
    pltpu.sync_copy(acc.at[pl.ds(r, CCH)], db0)
    pltpu.sync_copy(src.at[pl.ds(ar, CCH)], db1)
    _ew_loop(db0, db0, db1, CCH, lambda s, e: s + e * s)
    pltpu.sync_copy(db0, gcf_out.at[pl.ds(ar, CCH)])

    @pl.when(r >= N_OVERLAP)
    def _():
        if layer == 0:
            _ew_loop(db1, db1, db0, CCH, lambda x, y: x + y)
            pltpu.sync_copy(db1, sum_out.at[pl.ds(ar, CCH)])
        else:
            pltpu.sync_copy(sum_in.at[pl.ds(ar, CCH)], db2)
            _ew_loop(db2, db2, db0, CCH, lambda x, y: x + y)
            pltpu.sync_copy(db2, sum_out.at[pl.ds(ar, CCH)])


def _transfer_chunk(r, kc, layer, srcA, srcB, sumA_in, sumB_in,
                    gcfA, gcfB, sumA_out, sumB_out, db0, db1, db2):
    ar = kc * N + r
    pltpu.sync_copy(gcfA.at[pl.ds(ar, CCH)], db0)
    pltpu.sync_copy(gcfB.at[pl.ds(ar, CCH)], db1)

    def mix(i, _):
        rr = i // (HALF // 16)
        h = (i % (HALF // 16)) * 16
        a = db0[rr, pl.ds(h, 16)]
        b = db1[rr, pl.ds(h, 16)]
        db0[rr, pl.ds(h, 16)] = 0.6 * a + 0.4 * b
        db1[rr, pl.ds(h, 16)] = 0.6 * b + 0.4 * a
        return 0
    lax.fori_loop(0, CCH * HALF // 16, mix, 0)
    pltpu.sync_copy(db0, gcfA.at[pl.ds(ar, CCH)])
    pltpu.sync_copy(db1, gcfB.at[pl.ds(ar, CCH)])
    for sin, sout, dnew in ((srcA if layer == 0 else sumA_in, sumA_out, db0),
                            (srcB if layer == 0 else sumB_in, sumB_out, db1)):
        pltpu.sync_copy(sin.at[pl.ds(ar, CCH)], db2)
        _ew_loop(db2, db2, dnew, CCH, lambda x, y: x + y)
        pltpu.sync_copy(db2, sout.at[pl.ds(ar, CCH)])


def _make_layer_kernel(layer):
    eg = jax.ShapeDtypeStruct((NC * N, HALF), jnp.float32)

    @functools.partial(
        pl.kernel,
        out_type=(eg, eg, eg, eg),
        mesh=_MESH,
        compiler_params=_SC_PARAMS,
        scratch_types=[
            pltpu.VMEM_SHARED((N, HALF), jnp.float32),
            pltpu.VMEM((2, SUP, 2, CHUNK), jnp.int32),
            pltpu.VMEM((2, SUP, CHUNK), jnp.float32),
            pltpu.VMEM((2, SUP * CHUNK, HALF), jnp.float32),
            pltpu.VMEM((CCH, HALF), jnp.float32),
            pltpu.SemaphoreType.DMA,
            pltpu.SemaphoreType.DMA,
            pltpu.SemaphoreType.DMA,
            pltpu.SemaphoreType.DMA,
            pltpu.SemaphoreType.DMA,
            pltpu.SemaphoreType.DMA,
        ],
    )
    def layer_kernel(egoA, egoB, pkA, valsA, pkB, valsB,
                     sumA_in, sumB_in, zeros_hbm,
                     gcfA, gcfB, sumA_out, sumB_out,
                     acc, pb, vb, db, db2,
                     isem0, isem1, gsem0, gsem1, ssem0, ssem1):
        kc = lax.axis_index("c")
        t = lax.axis_index("s")
        isems = (isem0, isem1)
        gsems = (gsem0, gsem1)
        ssems = (ssem0, ssem1)
        db0 = db.at[0, pl.ds(0, CCH)]
        db1 = db.at[1, pl.ds(0, CCH)]

        for (src, pk2, vals2, gcf, sin, sout) in (
                (egoA, pkA, valsA, gcfA, sumA_in, sumA_out),
                (egoB, pkB, valsB, gcfB, sumB_in, sumB_out)):
            _zero_acc(zeros_hbm, acc, t)
            plsc.subcore_barrier()
            _spmm_phase(kc, t, src, pk2, vals2, zeros_hbm,
                        acc, pb, vb, db, isems, gsems, ssems)
            plsc.subcore_barrier()

            def comb(i, _, base):
                _combine_chunk(base + i * CCH, kc, layer, acc, src, sin,
                               gcf, sout, db0, db1, db2)
                return 0
            lax.fori_loop(0, RPT // CCH,
                          functools.partial(comb, base=t * RPT), 0)

            @pl.when(t < 2)
            def _():
                lax.fori_loop(
                    0, 1000 // CCH,
                    functools.partial(comb, base=NS * RPT + t * 1000), 0)
            plsc.subcore_barrier()

        def trans(i, _, base):
            _transfer_chunk(base + i * CCH, kc, layer, egoA, egoB,
                            sumA_in, sumB_in, gcfA, gcfB, sumA_out, sumB_out,
                            db0, db1, db2)
            return 0
        lax.fori_loop(0, 3, functools.partial(trans, base=t * 600), 0)

        @pl.when(t < 2)
        def _():
            lax.fori_loop(0, 1,
                          functools.partial(trans, base=NS * 600 + t * 200), 0)

    return layer_kernel


_layer_first = _make_layer_kernel(0)
_layer_rest = _make_layer_kernel(1)


def _make_gather_kernel():
    @functools.partial(
        pl.kernel,
        out_type=jax.ShapeDtypeStruct((2, 3, NC, BATCH, HALF), jnp.float32),
        mesh=_MESH,
        compiler_params=_SC_PARAMS,
        scratch_types=[
            pltpu.VMEM((2, CHUNK), jnp.int32),
            pltpu.VMEM((SPT, HALF), jnp.float32),
            pltpu.SemaphoreType.DMA,
        ],
    )
    def gather_kernel(sumA, sumB, datb, out, ib, gb, sem):
        kc = lax.axis_index("c")
        t = lax.axis_index("s")
        for d, src in ((0, sumA), (1, sumB)):
            for kind in range(3):
                pltpu.sync_copy(datb.at[d, kind, kc, pl.ds(t * 2, 2)], ib)
                for b in range(2):
                    pltpu.async_copy(src.at[ib.at[b]],
                                     gb.at[pl.ds(b * CHUNK, CHUNK)],
                                     sem).wait()
                pltpu.sync_copy(gb, out.at[d, kind, kc, pl.ds(t * SPT, SPT)])

    return gather_kernel


_gather_k = _make_gather_kernel()


def _loss_body(g_ref, out_ref):
    g = g_ref[...]
    u = g[:, 0]
    p = g[:, 1]
    n = g[:, 2]
    pos = jnp.sum(u * p, axis=(1, 3)) / 16.0
    neg = jnp.sum(u * n, axis=(1, 3)) / 16.0
    per = jnp.mean(jax.nn.softplus(neg - pos), axis=1)
    out_ref[0, 0] = per[0] + per[1]


def _pad_edges(idx, val):
    pad = NNZ_PAD - NNZ
    spread = (jnp.arange(pad, dtype=jnp.int32) * 64) % N
    cols = jnp.concatenate([idx[1].astype(jnp.int32), spread])
    rows = jnp.concatenate([idx[0].astype(jnp.int32), spread])
    vals = jnp.concatenate([val, jnp.zeros((pad,), jnp.float32)])
    one = jnp.stack([cols.reshape(NROWS2D, CHUNK),
                     rows.reshape(NROWS2D, CHUNK)], axis=1)
    pk = jnp.stack([one, one.at[:, 0].add(N)], axis=0)
    return pk, vals.reshape(NROWS2D, CHUNK)


def kernel(user_emb_a, item_emb_a, user_emb_b, item_emb_b,
           adj_a_val, adj_b_val, adj_a_idx, adj_b_idx, data_a, data_b):
    egoA = jnp.concatenate(
        [jnp.concatenate([user_emb_a[:, :HALF], item_emb_a[:, :HALF]]),
         jnp.concatenate([user_emb_a[:, HALF:], item_emb_a[:, HALF:]])])
    egoB = jnp.concatenate(
        [jnp.concatenate([user_emb_b[:, :HALF], item_emb_b[:, :HALF]]),
         jnp.concatenate([user_emb_b[:, HALF:], item_emb_b[:, HALF:]])])
    pkA, valsA = _pad_edges(adj_a_idx, adj_a_val)
    pkB, valsB = _pad_edges(adj_b_idx, adj_b_val)
    dat = jnp.stack([data_a.astype(jnp.int32), data_b.astype(jnp.int32)])
    kind_bias = jnp.array([0, N_USER, N_USER], jnp.int32)[None, :, None]
    core_bias = jnp.array([0, N], jnp.int32)[None, None, :, None]
    datb = (dat + kind_bias)[:, :, None, :] + core_bias
    datb = datb.reshape(2, 3, NC, BATCH // CHUNK, CHUNK)
    zeros = jnp.zeros((RPT, HALF), jnp.float32)

    sumA = sumB = jnp.zeros((NC * N, HALF), jnp.float32)
    for layer in range(LAYERS):
        fn = _layer_first if layer == 0 else _layer_rest
        egoA, egoB, sumA, sumB = fn(egoA, egoB, pkA, valsA, pkB, valsB,
                                    sumA, sumB, zeros)
    gbuf = _gather_k(sumA, sumB, datb)
    loss = pl.pallas_call(
        _loss_body,
        out_shape=jax.ShapeDtypeStruct((1, 1), jnp.float32),
        out_specs=pl.BlockSpec(memory_space=pltpu.SMEM),
    )(gbuf)
    return loss[0, 0]


# speedup vs baseline: 1.0055x; 1.0055x over previous
import functools

import jax
import jax.numpy as jnp
from jax import lax
from jax.experimental import pallas as pl
from jax.experimental.pallas import tpu as pltpu
from jax.experimental.pallas import tpu_sc as plsc

N_USER = 25000
N_OVERLAP = 10000
EMB = 64
LAYERS = 3
N = 50000
BATCH = 4096

NNZ = 800000
HALF = 32
CHUNK = 128
NS = 16
NC = 2
SUP = 2
NSTEP = 200
NCT = SUP * NSTEP
NNZ_PAD = NS * NCT * CHUNK
NROWS2D = NNZ_PAD // CHUNK
RPT = 3000
CCH = 200
SPT = BATCH // NS

_MESH = plsc.VectorSubcoreMesh(core_axis_name="c", subcore_axis_name="s",
                               num_cores=NC, num_subcores=NS)
_SC_PARAMS = pltpu.CompilerParams(use_tc_tiling_on_sc=False)


def _zero_acc(zeros_hbm, acc, t):
    pltpu.sync_copy(zeros_hbm, acc.at[pl.ds(t * RPT, RPT)])

    @pl.when(t < 2)
    def _():
        rx = NS * RPT + t * 1000
        pltpu.sync_copy(zeros_hbm.at[pl.ds(0, 1000)], acc.at[pl.ds(rx, 1000)])


def _spmm_phase(kc, t, ego_hbm, pk2, vals2, zeros_hbm,
                acc, pb, vb, db, isems, gsems, ssems):
    def idx_load(p, s):
        row = t * NCT + s * SUP
        return [
            pltpu.async_copy(pk2.at[kc, pl.ds(row, SUP)], pb.at[p], isems[p]),
            pltpu.async_copy(vals2.at[pl.ds(row, SUP)], vb.at[p], isems[p]),
        ]

    def gathers(p):
        for b in range(SUP):
            pltpu.async_copy(ego_hbm.at[pb.at[p, b, 0]],
                             db.at[p, pl.ds(b * CHUNK, CHUNK)], gsems[p])

    def drain(sem):
        pltpu.make_async_copy(zeros_hbm.at[pl.ds(0, SUP * CHUNK)],
                              db.at[0], sem).wait()

    def scale(p):
        def grp(g, _):
            b = g // (CHUNK // 32)
            o = (g % (CHUNK // 32)) * 32
            for half in range(2):
                v = vb[p, b, pl.ds(o + half * 16, 16)]
                for j in range(16):
                    e = b * CHUNK + o + half * 16 + j
                    s = v[j]
                    db[p, e, pl.ds(0, 16)] = db[p, e, pl.ds(0, 16)] * s
                    db[p, e, pl.ds(16, 16)] = db[p, e, pl.ds(16, 16)] * s
            return 0
        lax.fori_loop(0, SUP * CHUNK // 32, grp, 0)

    def scatters(p):
        for b in range(SUP):
            pltpu.async_copy(db.at[p, pl.ds(b * CHUNK, CHUNK)],
                             acc.at[pb.at[p, b, 1]], ssems[p], add=True)

    for d in idx_load(0, 0):
        d.wait()
    gathers(0)
    for d in idx_load(1, 1):
        d.wait()

    def body2(s, _):
        @pl.when(s > 0)
        def _():
            drain(ssems[1])
        gathers(1)
        drain(gsems[0])
        di = idx_load(0, s + 2)
        scale(0)
        scatters(0)
        for d in di:
            d.wait()
        drain(ssems[0])
        gathers(0)
        drain(gsems[1])
        di = idx_load(1, s + 3)
        scale(1)
        scatters(1)
        for d in di:
            d.wait()
        return 0
    lax.fori_loop(0, (NSTEP - 2) // 2, lambda i, c: body2(i * 2, c), 0)

    drain(ssems[1])
    gathers(1)
    drain(gsems[0])
    scale(0)
    scatters(0)
    drain(gsems[1])
    scale(1)
    scatters(1)
    drain(ssems[0])
    drain(ssems[1])


def _ew_loop(dst, a, b, nrows, f):
    def body(i, _):
        r = i // (HALF // 16)
        h = (i % (HALF // 16)) * 16
        dst[r, pl.ds(h, 16)] = f(a[r, pl.ds(h, 16)], b[r, pl.ds(h, 16)])
        return 0
    lax.fori_loop(0, nrows * HALF // 16, body, 0)


def _combine_chunk(r, kc, layer, acc, src, sum_in, gcf_out, sum_out,
                   db0, db1, db2):
    ar = kc * N + r
    pltpu.sync_copy(acc.at[pl.ds(r, CCH)], db0)
    pltpu.sync_copy(src.at[pl.ds(ar, CCH)], db1)
    _ew_loop(db0, db0, db1, CCH, lambda s, e: s + e * s)
    pltpu.sync_copy(db0, gcf_out.at[pl.ds(ar, CCH)])

    @pl.when(r >= N_OVERLAP)
    def _():
        if layer == 0:
            _ew_loop(db1, db1, db0, CCH, lambda x, y: x + y)
            pltpu.sync_copy(db1, sum_out.at[pl.ds(ar, CCH)])
        else:
            pltpu.sync_copy(sum_in.at[pl.ds(ar, CCH)], db2)
            _ew_loop(db2, db2, db0, CCH, lambda x, y: x + y)
            pltpu.sync_copy(db2, sum_out.at[pl.ds(ar, CCH)])


def _transfer_chunk(r, kc, layer, srcA, srcB, sumA_in, sumB_in,
                    gcfA, gcfB, sumA_out, sumB_out, db0, db1, db2):
    ar = kc * N + r
    pltpu.sync_copy(gcfA.at[pl.ds(ar, CCH)], db0)
    pltpu.sync_copy(gcfB.at[pl.ds(ar, CCH)], db1)

    def mix(i, _):
        rr = i // (HALF // 16)
        h = (i % (HALF // 16)) * 16
        a = db0[rr, pl.ds(h, 16)]
        b = db1[rr, pl.ds(h, 16)]
        db0[rr, pl.ds(h, 16)] = 0.6 * a + 0.4 * b
        db1[rr, pl.ds(h, 16)] = 0.6 * b + 0.4 * a
        return 0
    lax.fori_loop(0, CCH * HALF // 16, mix, 0)
    pltpu.sync_copy(db0, gcfA.at[pl.ds(ar, CCH)])
    pltpu.sync_copy(db1, gcfB.at[pl.ds(ar, CCH)])
    for sin, sout, dnew in ((srcA if layer == 0 else sumA_in, sumA_out, db0),
                            (srcB if layer == 0 else sumB_in, sumB_out, db1)):
        pltpu.sync_copy(sin.at[pl.ds(ar, CCH)], db2)
        _ew_loop(db2, db2, dnew, CCH, lambda x, y: x + y)
        pltpu.sync_copy(db2, sout.at[pl.ds(ar, CCH)])


def _make_mega_kernel():
    eg = jax.ShapeDtypeStruct((NC * N, HALF), jnp.float32)
    gb_t = jax.ShapeDtypeStruct((2, 3, NC, BATCH, HALF), jnp.float32)

    @functools.partial(
        pl.kernel,
        out_type=(eg, eg, eg, eg, eg, eg, eg, eg, gb_t),
        mesh=_MESH,
        compiler_params=_SC_PARAMS,
        scratch_types=[
            pltpu.VMEM_SHARED((N, HALF), jnp.float32),
            pltpu.VMEM((2, SUP, 2, CHUNK), jnp.int32),
            pltpu.VMEM((2, SUP, CHUNK), jnp.float32),
            pltpu.VMEM((2, SUP * CHUNK, HALF), jnp.float32),
            pltpu.VMEM((CCH, HALF), jnp.float32),
            pltpu.SemaphoreType.DMA,
            pltpu.SemaphoreType.DMA,
            pltpu.SemaphoreType.DMA,
            pltpu.SemaphoreType.DMA,
            pltpu.SemaphoreType.DMA,
            pltpu.SemaphoreType.DMA,
        ],
    )
    def mega_kernel(egoA, egoB, pkA, valsA, pkB, valsB, datb, zeros_hbm,
                    e1A, e2A, s0A, s1A, e1B, e2B, s0B, s1B, gbuf,
                    acc, pb, vb, db, db2,
                    isem0, isem1, gsem0, gsem1, ssem0, ssem1):
        kc = lax.axis_index("c")
        t = lax.axis_index("s")
        isems = (isem0, isem1)
        gsems = (gsem0, gsem1)
        ssems = (ssem0, ssem1)
        db0 = db.at[0, pl.ds(0, CCH)]
        db1 = db.at[1, pl.ds(0, CCH)]

        plan = (
            (egoA, e1A, None, s0A, egoB, e1B, None, s0B),
            (e1A, e2A, s0A, s1A, e1B, e2B, s0B, s1B),
            (e2A, e1A, s1A, s0A, e2B, e1B, s1B, s0B),
        )
        for layer, (srcA, gcfA, sinA, soutA,
                    srcB, gcfB, sinB, soutB) in enumerate(plan):
            lflag = 0 if layer == 0 else 1
            for (src, pk2, vals2, gcf, sin, sout) in (
                    (srcA, pkA, valsA, gcfA, sinA, soutA),
                    (srcB, pkB, valsB, gcfB, sinB, soutB)):
                _zero_acc(zeros_hbm, acc, t)
                plsc.subcore_barrier()
                _spmm_phase(kc, t, src, pk2, vals2, zeros_hbm,
                            acc, pb, vb, db, isems, gsems, ssems)
                plsc.subcore_barrier()

                def comb(i, _, base, args=(src, sin, gcf, sout)):
                    s_, si_, g_, so_ = args
                    _combine_chunk(base + i * CCH, kc, lflag, acc, s_, si_,
                                   g_, so_, db0, db1, db2)
                    return 0
                lax.fori_loop(0, RPT // CCH,
                              functools.partial(comb, base=t * RPT), 0)

                @pl.when(t < 2)
                def _():
                    lax.fori_loop(
                        0, 1000 // CCH,
                        functools.partial(comb, base=NS * RPT + t * 1000), 0)
                plsc.subcore_barrier()

            def trans(i, _, base,
                      args=(srcA, srcB, sinA, sinB, gcfA, gcfB, soutA, soutB)):
                sa, sb, ia, ib_, ga, gb_, oa, ob = args
                _transfer_chunk(base + i * CCH, kc, lflag, sa, sb, ia, ib_,
                                ga, gb_, oa, ob, db0, db1, db2)
                return 0
            lax.fori_loop(0, 3, functools.partial(trans, base=t * 600), 0)

            @pl.when(t < 2)
            def _():
                lax.fori_loop(
                    0, 1, functools.partial(trans, base=NS * 600 + t * 200), 0)
            plsc.subcore_barrier()

        for d, ssrc in ((0, s0A), (1, s0B)):
            for kind in range(3):
                pltpu.sync_copy(datb.at[d, kind, kc, pl.ds(t * 2, 2)],
                                pb.at[0, 0])
                for b in range(2):
                    pltpu.async_copy(ssrc.at[pb.at[0, 0, b]],
                                     db.at[0, pl.ds(b * CHUNK, CHUNK)],
                                     gsem0).wait()
                pltpu.sync_copy(db.at[0, pl.ds(0, SPT)],
                                gbuf.at[d, kind, kc, pl.ds(t * SPT, SPT)])

    return mega_kernel


_mega_k = _make_mega_kernel()


def _loss_body(g_ref, out_ref):
    g = g_ref[...]
    u = g[:, 0]
    p = g[:, 1]
    n = g[:, 2]
    pos = jnp.sum(u * p, axis=(1, 3)) / 16.0
    neg = jnp.sum(u * n, axis=(1, 3)) / 16.0
    per = jnp.mean(jax.nn.softplus(neg - pos), axis=1)
    out_ref[0, 0] = per[0] + per[1]


def _pad_edges(idx, val):
    pad = NNZ_PAD - NNZ
    spread = (jnp.arange(pad, dtype=jnp.int32) * 64) % N
    cols = jnp.concatenate([idx[1].astype(jnp.int32), spread])
    rows = jnp.concatenate([idx[0].astype(jnp.int32), spread])
    vals = jnp.concatenate([val, jnp.zeros((pad,), jnp.float32)])
    one = jnp.stack([cols.reshape(NROWS2D, CHUNK),
                     rows.reshape(NROWS2D, CHUNK)], axis=1)
    pk = jnp.stack([one, one.at[:, 0].add(N)], axis=0)
    return pk, vals.reshape(NROWS2D, CHUNK)


def kernel(user_emb_a, item_emb_a, user_emb_b, item_emb_b,
           adj_a_val, adj_b_val, adj_a_idx, adj_b_idx, data_a, data_b):
    egoA = jnp.concatenate(
        [jnp.concatenate([user_emb_a[:, :HALF], item_emb_a[:, :HALF]]),
         jnp.concatenate([user_emb_a[:, HALF:], item_emb_a[:, HALF:]])])
    egoB = jnp.concatenate(
        [jnp.concatenate([user_emb_b[:, :HALF], item_emb_b[:, :HALF]]),
         jnp.concatenate([user_emb_b[:, HALF:], item_emb_b[:, HALF:]])])
    pkA, valsA = _pad_edges(adj_a_idx, adj_a_val)
    pkB, valsB = _pad_edges(adj_b_idx, adj_b_val)
    dat = jnp.stack([data_a.astype(jnp.int32), data_b.astype(jnp.int32)])
    kind_bias = jnp.array([0, N_USER, N_USER], jnp.int32)[None, :, None]
    core_bias = jnp.array([0, N], jnp.int32)[None, None, :, None]
    datb = (dat + kind_bias)[:, :, None, :] + core_bias
    datb = datb.reshape(2, 3, NC, BATCH // CHUNK, CHUNK)
    zeros = jnp.zeros((RPT, HALF), jnp.float32)

    gbuf = _mega_k(egoA, egoB, pkA, valsA, pkB, valsB, datb, zeros)[-1]
    loss = pl.pallas_call(
        _loss_body,
        out_shape=jax.ShapeDtypeStruct((1, 1), jnp.float32),
        out_specs=pl.BlockSpec(memory_space=pltpu.SMEM),
    )(gbuf)
    return loss[0, 0]


# speedup vs baseline: 1.1557x; 1.1494x over previous
import functools

import jax
import jax.numpy as jnp
from jax import lax
from jax.experimental import pallas as pl
from jax.experimental.pallas import tpu as pltpu
from jax.experimental.pallas import tpu_sc as plsc

N_USER = 25000
N_OVERLAP = 10000
EMB = 64
LAYERS = 3
N = 50000
BATCH = 4096

NNZ = 800000
HALF = 32
CHUNK = 128
NS = 16
NC = 2
SUP = 2
NSTEP = 200
NCT = SUP * NSTEP
NNZ_PAD = NS * NCT * CHUNK
NROWS2D = NNZ_PAD // CHUNK
RPT = 3000
CCH = 200
SPT = BATCH // NS

_MESH = plsc.VectorSubcoreMesh(core_axis_name="c", subcore_axis_name="s",
                               num_cores=NC, num_subcores=NS)
_SC_PARAMS = pltpu.CompilerParams(use_tc_tiling_on_sc=False)


def _zero_acc(zeros_hbm, acc, t):
    pltpu.sync_copy(zeros_hbm, acc.at[pl.ds(t * RPT, RPT)])

    @pl.when(t < 2)
    def _():
        rx = NS * RPT + t * 1000
        pltpu.sync_copy(zeros_hbm.at[pl.ds(0, 1000)], acc.at[pl.ds(rx, 1000)])


def _spmm_phase(kc, t, ego_hbm, cols2, rows2, vals2, zeros_hbm,
                acc, cb, rb, vb, db, isems, gsems, ssems):
    def idx_load(p, s):
        row = t * NCT + s * SUP
        sl = pl.ds(row, SUP)
        return [
            pltpu.async_copy(cols2.at[kc, sl], cb.at[p], isems[p]),
            pltpu.async_copy(rows2.at[sl], rb.at[p], isems[p]),
            pltpu.async_copy(vals2.at[sl], vb.at[p], isems[p]),
        ]

    def gathers(p):
        for b in range(SUP):
            pltpu.async_copy(ego_hbm.at[cb.at[p, b]],
                             db.at[p, pl.ds(b * CHUNK, CHUNK)], gsems[p])

    def drain(sem):
        for _ in range(SUP):
            pltpu.make_async_copy(zeros_hbm.at[pl.ds(0, CHUNK)],
                                  db.at[0, pl.ds(0, CHUNK)], sem).wait()

    def scale(p):
        def grp(g, _):
            v = vb[p, g // (CHUNK // 16), pl.ds((g % (CHUNK // 16)) * 16, 16)]
            for j in range(16):
                e = g * 16 + j
                s = v[j]
                db[p, e, pl.ds(0, 16)] = db[p, e, pl.ds(0, 16)] * s
                db[p, e, pl.ds(16, 16)] = db[p, e, pl.ds(16, 16)] * s
            return 0
        lax.fori_loop(0, SUP * CHUNK // 16, grp, 0)

    def scatters(p):
        for b in range(SUP):
            pltpu.async_copy(db.at[p, pl.ds(b * CHUNK, CHUNK)],
                             acc.at[rb.at[p, b]], ssems[p], add=True)

    for d in idx_load(0, 0):
        d.wait()
    gathers(0)
    for d in idx_load(1, 1):
        d.wait()

    def body2(s, _):
        @pl.when(s > 0)
        def _():
            drain(ssems[1])
        gathers(1)
        drain(gsems[0])
        di = idx_load(0, s + 2)
        scale(0)
        scatters(0)
        for d in di:
            d.wait()
        drain(ssems[0])
        gathers(0)
        drain(gsems[1])
        di = idx_load(1, s + 3)
        scale(1)
        scatters(1)
        for d in di:
            d.wait()
        return 0
    lax.fori_loop(0, (NSTEP - 2) // 2, lambda i, c: body2(i * 2, c), 0)

    drain(ssems[1])
    gathers(1)
    drain(gsems[0])
    scale(0)
    scatters(0)
    drain(gsems[1])
    scale(1)
    scatters(1)
    drain(ssems[0])
    drain(ssems[1])


def _ew_loop(dst, a, b, nrows, f):
    def body(i, _):
        r = i // (HALF // 16)
        h = (i % (HALF // 16)) * 16
        dst[r, pl.ds(h, 16)] = f(a[r, pl.ds(h, 16)], b[r, pl.ds(h, 16)])
        return 0
    lax.fori_loop(0, nrows * HALF // 16, body, 0)


def _combine_chunk(r, kc, layer, acc, src, sum_in, gcf_out, sum_out,
                   db0, db1, db2):
    ar = kc * N + r
    pltpu.sync_copy(acc.at[pl.ds(r, CCH)], db0)
    pltpu.sync_copy(src.at[pl.ds(ar, CCH)], db1)
    _ew_loop(db0, db0, db1, CCH, lambda s, e: s + e * s)
    pltpu.sync_copy(db0, gcf_out.at[pl.ds(ar, CCH)])

    @pl.when(r >= N_OVERLAP)
    def _():
        if layer == 0:
            _ew_loop(db1, db1, db0, CCH, lambda x, y: x + y)
            pltpu.sync_copy(db1, sum_out.at[pl.ds(ar, CCH)])
        else:
            pltpu.sync_copy(sum_in.at[pl.ds(ar, CCH)], db2)
            _ew_loop(db2, db2, db0, CCH, lambda x, y: x + y)
            pltpu.sync_copy(db2, sum_out.at[pl.ds(ar, CCH)])


def _transfer_chunk(r, kc, layer, srcA, srcB, sumA_in, sumB_in,
                    gcfA, gcfB, sumA_out, sumB_out, db0, db1, db2):
    ar = kc * N + r
    pltpu.sync_copy(gcfA.at[pl.ds(ar, CCH)], db0)
    pltpu.sync_copy(gcfB.at[pl.ds(ar, CCH)], db1)

    def mix(i, _):
        rr = i // (HALF // 16)
        h = (i % (HALF // 16)) * 16
        a = db0[rr, pl.ds(h, 16)]
        b = db1[rr, pl.ds(h, 16)]
        db0[rr, pl.ds(h, 16)] = 0.6 * a + 0.4 * b
        db1[rr, pl.ds(h, 16)] = 0.6 * b + 0.4 * a
        return 0
    lax.fori_loop(0, CCH * HALF // 16, mix, 0)
    pltpu.sync_copy(db0, gcfA.at[pl.ds(ar, CCH)])
    pltpu.sync_copy(db1, gcfB.at[pl.ds(ar, CCH)])
    for sin, sout, dnew in ((srcA if layer == 0 else sumA_in, sumA_out, db0),
                            (srcB if layer == 0 else sumB_in, sumB_out, db1)):
        pltpu.sync_copy(sin.at[pl.ds(ar, CCH)], db2)
        _ew_loop(db2, db2, dnew, CCH, lambda x, y: x + y)
        pltpu.sync_copy(db2, sout.at[pl.ds(ar, CCH)])


def _make_mega_kernel():
    eg = jax.ShapeDtypeStruct((NC * N, HALF), jnp.float32)
    gb_t = jax.ShapeDtypeStruct((2, 3, NC, BATCH, HALF), jnp.float32)

    @functools.partial(
        pl.kernel,
        out_type=(eg, eg, eg, eg, eg, eg, eg, eg, gb_t),
        mesh=_MESH,
        compiler_params=_SC_PARAMS,
        scratch_types=[
            pltpu.VMEM_SHARED((N, HALF), jnp.float32),
            pltpu.VMEM((2, SUP, CHUNK), jnp.int32),
            pltpu.VMEM((2, SUP, CHUNK), jnp.int32),
            pltpu.VMEM((2, SUP, CHUNK), jnp.float32),
            pltpu.VMEM((2, SUP * CHUNK, HALF), jnp.float32),
            pltpu.VMEM((CCH, HALF), jnp.float32),
            pltpu.SemaphoreType.DMA,
            pltpu.SemaphoreType.DMA,
            pltpu.SemaphoreType.DMA,
            pltpu.SemaphoreType.DMA,
            pltpu.SemaphoreType.DMA,
            pltpu.SemaphoreType.DMA,
        ],
    )
    def mega_kernel(egoA, egoB, colsA, rowsA, valsA, colsB, rowsB, valsB,
                    datb, zeros_hbm,
                    e1A, e2A, s0A, s1A, e1B, e2B, s0B, s1B, gbuf,
                    acc, cb, rb, vb, db, db2,
                    isem0, isem1, gsem0, gsem1, ssem0, ssem1):
        kc = lax.axis_index("c")
        t = lax.axis_index("s")
        isems = (isem0, isem1)
        gsems = (gsem0, gsem1)
        ssems = (ssem0, ssem1)
        db0 = db.at[0, pl.ds(0, CCH)]
        db1 = db.at[1, pl.ds(0, CCH)]

        plan = (
            (egoA, e1A, None, s0A, egoB, e1B, None, s0B),
            (e1A, e2A, s0A, s1A, e1B, e2B, s0B, s1B),
            (e2A, e1A, s1A, s0A, e2B, e1B, s1B, s0B),
        )
        for layer, (srcA, gcfA, sinA, soutA,
                    srcB, gcfB, sinB, soutB) in enumerate(plan):
            lflag = 0 if layer == 0 else 1
            for (src, c2, r2, v2, gcf, sin, sout) in (
                    (srcA, colsA, rowsA, valsA, gcfA, sinA, soutA),
                    (srcB, colsB, rowsB, valsB, gcfB, sinB, soutB)):
                _zero_acc(zeros_hbm, acc, t)
                plsc.subcore_barrier()
                _spmm_phase(kc, t, src, c2, r2, v2, zeros_hbm,
                            acc, cb, rb, vb, db, isems, gsems, ssems)
                plsc.subcore_barrier()

                def comb(i, _, base, args=(src, sin, gcf, sout)):
                    s_, si_, g_, so_ = args
                    _combine_chunk(base + i * CCH, kc, lflag, acc, s_, si_,
                                   g_, so_, db0, db1, db2)
                    return 0
                lax.fori_loop(0, RPT // CCH,
                              functools.partial(comb, base=t * RPT), 0)

                @pl.when(t < 2)
                def _():
                    lax.fori_loop(
                        0, 1000 // CCH,
                        functools.partial(comb, base=NS * RPT + t * 1000), 0)
                plsc.subcore_barrier()

            def trans(i, _, base,
                      args=(srcA, srcB, sinA, sinB, gcfA, gcfB, soutA, soutB)):
                sa, sb, ia, ib_, ga, gb_, oa, ob = args
                _transfer_chunk(base + i * CCH, kc, lflag, sa, sb, ia, ib_,
                                ga, gb_, oa, ob, db0, db1, db2)
                return 0
            lax.fori_loop(0, 3, functools.partial(trans, base=t * 600), 0)

            @pl.when(t < 2)
            def _():
                lax.fori_loop(
                    0, 1, functools.partial(trans, base=NS * 600 + t * 200), 0)
            plsc.subcore_barrier()

        for d, ssrc in ((0, s0A), (1, s0B)):
            for kind in range(3):
                pltpu.sync_copy(datb.at[d, kind, kc, pl.ds(t * 2, 2)],
                                cb.at[0])
                for b in range(2):
                    pltpu.async_copy(ssrc.at[cb.at[0, b]],
                                     db.at[0, pl.ds(b * CHUNK, CHUNK)],
                                     gsem0).wait()
                pltpu.sync_copy(db.at[0, pl.ds(0, SPT)],
                                gbuf.at[d, kind, kc, pl.ds(t * SPT, SPT)])

    return mega_kernel


_mega_k = _make_mega_kernel()


def _loss_body(g_ref, out_ref):
    g = g_ref[...]
    u = g[:, 0]
    p = g[:, 1]
    n = g[:, 2]
    pos = jnp.sum(u * p, axis=(1, 3)) / 16.0
    neg = jnp.sum(u * n, axis=(1, 3)) / 16.0
    per = jnp.mean(jax.nn.softplus(neg - pos), axis=1)
    out_ref[0, 0] = per[0] + per[1]


def _pad_edges(idx, val):
    pad = NNZ_PAD - NNZ
    spread = (jnp.arange(pad, dtype=jnp.int32) * 64) % N
    cols = jnp.concatenate([idx[1].astype(jnp.int32), spread])
    rows = jnp.concatenate([idx[0].astype(jnp.int32), spread])
    vals = jnp.concatenate([val, jnp.zeros((pad,), jnp.float32)])
    cols2 = jnp.stack([cols, cols + N]).reshape(NC, NROWS2D, CHUNK)
    return cols2, rows.reshape(NROWS2D, CHUNK), vals.reshape(NROWS2D, CHUNK)


def kernel(user_emb_a, item_emb_a, user_emb_b, item_emb_b,
           adj_a_val, adj_b_val, adj_a_idx, adj_b_idx, data_a, data_b):
    egoA = jnp.concatenate(
        [jnp.concatenate([user_emb_a[:, :HALF], item_emb_a[:, :HALF]]),
         jnp.concatenate([user_emb_a[:, HALF:], item_emb_a[:, HALF:]])])
    egoB = jnp.concatenate(
        [jnp.concatenate([user_emb_b[:, :HALF], item_emb_b[:, :HALF]]),
         jnp.concatenate([user_emb_b[:, HALF:], item_emb_b[:, HALF:]])])
    colsA, rowsA, valsA = _pad_edges(adj_a_idx, adj_a_val)
    colsB, rowsB, valsB = _pad_edges(adj_b_idx, adj_b_val)
    dat = jnp.stack([data_a.astype(jnp.int32), data_b.astype(jnp.int32)])
    kind_bias = jnp.array([0, N_USER, N_USER], jnp.int32)[None, :, None]
    core_bias = jnp.array([0, N], jnp.int32)[None, None, :, None]
    datb = (dat + kind_bias)[:, :, None, :] + core_bias
    datb = datb.reshape(2, 3, NC, BATCH // CHUNK, CHUNK)
    zeros = jnp.zeros((RPT, HALF), jnp.float32)

    gbuf = _mega_k(egoA, egoB, colsA, rowsA, valsA, colsB, rowsB, valsB,
                   datb, zeros)[-1]
    loss = pl.pallas_call(
        _loss_body,
        out_shape=jax.ShapeDtypeStruct((1, 1), jnp.float32),
        out_specs=pl.BlockSpec(memory_space=pltpu.SMEM),
    )(gbuf)
    return loss[0, 0]
